# baseline (speedup 1.0000x reference)
import functools

import jax
import jax.numpy as jnp
from jax import lax
from jax.experimental import pallas as pl
from jax.experimental.pallas import tpu as pltpu
from jax.experimental.pallas import tpu_sc as plsc

MAX_OBJ = 100
PAD = 112
BASE = 96
SHIFT = 100 - BASE
SLAB = 120
B = 16
LANES = 16
LMK_CH = tuple(range(6, 21))


def _sel_batches(call_id):
    key = jax.random.fold_in(jax.random.key(42), call_id)
    return jnp.sort(jax.random.randint(key, (MAX_OBJ,), 0, B, dtype=jnp.int32))


def _sc_body(x_head, x_face, x_body, idx, o_head, o_face, o_body,
             slab6, slab21, out7, out22, idxv):
    wid = lax.axis_index("c") * 16 + lax.axis_index("s")

    def one_head(k, x_hbm, out_hbm, slab, outv, n_ch, lmks):
        @pl.when(wid == k)
        def _():
            pltpu.sync_copy(x_hbm.at[:, pl.ds(BASE, SLAB), :], slab)
            pltpu.sync_copy(idx.at[pl.ds(k * PAD, PAD)], idxv)
            for g in range(PAD // LANES):
                bvec = idxv[pl.ds(g * LANES, LANES)]
                ivec = lax.iota(jnp.int32, LANES) + g * LANES
                ch = [plsc.load_gather(
                          slab, [bvec, ivec + SHIFT,
                                 jnp.full((LANES,), c, jnp.int32)])
                      for c in range(n_ch)]
                cols = [bvec.astype(jnp.float32),
                        ch[0] - 0.5 * ch[2], ch[1] - 0.5 * ch[3],
                        ch[0] + 0.5 * ch[2], ch[1] + 0.5 * ch[3],
                        jnp.zeros((LANES,), jnp.float32),
                        ch[4] * ch[5]]
                cols += [ch[c] for c in lmks]
                for j, col in enumerate(cols):
                    plsc.store_scatter(
                        outv, [ivec, jnp.full((LANES,), j, jnp.int32)], col)
            pltpu.sync_copy(outv.at[pl.ds(0, MAX_OBJ), :], out_hbm)

    one_head(0, x_head, o_head, slab6, out7, 6, ())
    one_head(1, x_face, o_face, slab21, out22, 21, LMK_CH)
    one_head(2, x_body, o_body, slab6, out7, 6, ())


@jax.jit
def kernel(IDetectBody, IDetectHead, IKeypoint):
    idx = jnp.stack([_sel_batches(1), _sel_batches(2), _sel_batches(0)])
    idx = jnp.pad(idx, ((0, 0), (0, PAD - MAX_OBJ))).reshape(-1)

    f32 = jnp.float32
    call = pl.kernel(
        _sc_body,
        out_type=(jax.ShapeDtypeStruct((MAX_OBJ, 7), f32),
                  jax.ShapeDtypeStruct((MAX_OBJ, 22), f32),
                  jax.ShapeDtypeStruct((MAX_OBJ, 7), f32)),
        mesh=plsc.VectorSubcoreMesh(core_axis_name="c", subcore_axis_name="s"),
        compiler_params=pltpu.CompilerParams(needs_layout_passes=False,
                                             use_tc_tiling_on_sc=False),
        scratch_types=[
            pltpu.VMEM((B, SLAB, 6), f32),
            pltpu.VMEM((B, SLAB, 21), f32),
            pltpu.VMEM((PAD, 7), f32),
            pltpu.VMEM((PAD, 22), f32),
            pltpu.VMEM((PAD,), jnp.int32),
        ],
    )
    return call(IDetectHead, IKeypoint, IDetectBody, idx)


# speedup vs baseline: 12.0273x; 12.0273x over previous
import functools

import jax
import jax.numpy as jnp
from jax import lax
from jax.experimental import pallas as pl
from jax.experimental.pallas import tpu as pltpu
from jax.experimental.pallas import tpu_sc as plsc

MAX_OBJ = 100
PAD = 112
BASE = 96
SHIFT = 100 - BASE
SLAB = 120
B = 16
LANES = 16
LMK_CH = tuple(range(6, 21))


def _sel_batches(call_id):
    key = jax.random.fold_in(jax.random.key(42), call_id)
    return jnp.sort(jax.random.randint(key, (MAX_OBJ,), 0, B, dtype=jnp.int32))


def _sc_body(x_head, x_face, x_body, idx, o_head, o_face, o_body,
             slab6, slab21, out7, out22, idxv):
    wid = lax.axis_index("c") * 16 + lax.axis_index("s")

    def one_head(k, x_hbm, out_hbm, slab, outv, n_ch, lmks):
        @pl.when(wid == k)
        def _():
            pltpu.sync_copy(x_hbm, slab)
            pltpu.sync_copy(idx.at[pl.ds(k * PAD, PAD)], idxv)
            for g in range(PAD // LANES):
                bvec = idxv[pl.ds(g * LANES, LANES)]
                ivec = lax.iota(jnp.int32, LANES) + g * LANES
                ch = [plsc.load_gather(
                          slab, [bvec, ivec + SHIFT,
                                 jnp.full((LANES,), c, jnp.int32)])
                      for c in range(n_ch)]
                cols = [bvec.astype(jnp.float32),
                        ch[0] - 0.5 * ch[2], ch[1] - 0.5 * ch[3],
                        ch[0] + 0.5 * ch[2], ch[1] + 0.5 * ch[3],
                        jnp.zeros((LANES,), jnp.float32),
                        ch[4] * ch[5]]
                cols += [ch[c] for c in lmks]
                for j, col in enumerate(cols):
                    plsc.store_scatter(
                        outv, [ivec, jnp.full((LANES,), j, jnp.int32)], col)
            pltpu.sync_copy(outv.at[pl.ds(0, MAX_OBJ), :], out_hbm)

    one_head(0, x_head, o_head, slab6, out7, 6, ())
    one_head(1, x_face, o_face, slab21, out22, 21, LMK_CH)
    one_head(2, x_body, o_body, slab6, out7, 6, ())


@jax.jit
def kernel(IDetectBody, IDetectHead, IKeypoint):
    idx = jnp.stack([_sel_batches(1), _sel_batches(2), _sel_batches(0)])
    idx = jnp.pad(idx, ((0, 0), (0, PAD - MAX_OBJ))).reshape(-1)

    sb = lax.slice_in_dim(IDetectBody, BASE, BASE + SLAB, axis=1)
    sh = lax.slice_in_dim(IDetectHead, BASE, BASE + SLAB, axis=1)
    sf = lax.slice_in_dim(IKeypoint, BASE, BASE + SLAB, axis=1)

    f32 = jnp.float32
    call = pl.kernel(
        _sc_body,
        out_type=(jax.ShapeDtypeStruct((MAX_OBJ, 7), f32),
                  jax.ShapeDtypeStruct((MAX_OBJ, 22), f32),
                  jax.ShapeDtypeStruct((MAX_OBJ, 7), f32)),
        mesh=plsc.VectorSubcoreMesh(core_axis_name="c", subcore_axis_name="s"),
        compiler_params=pltpu.CompilerParams(needs_layout_passes=False,
                                             use_tc_tiling_on_sc=False),
        scratch_types=[
            pltpu.VMEM((B, SLAB, 6), f32),
            pltpu.VMEM((B, SLAB, 21), f32),
            pltpu.VMEM((PAD, 7), f32),
            pltpu.VMEM((PAD, 22), f32),
            pltpu.VMEM((PAD,), jnp.int32),
        ],
    )
    return call(sh, sf, sb, idx)


# speedup vs baseline: 22.4503x; 1.8666x over previous
import functools

import jax
import jax.numpy as jnp
from jax import lax
from jax.experimental import pallas as pl
from jax.experimental.pallas import tpu as pltpu
from jax.experimental.pallas import tpu_sc as plsc

MAX_OBJ = 100
PAD = 112
BASE = 96
SHIFT = 100 - BASE
SLAB = 120
B = 16
LANES = 16
LMK_CH = tuple(range(6, 21))


_SEL = {
    0: (0, 0, 0, 0, 0, 0, 0, 1, 1, 1, 1, 1, 2, 2, 2, 2, 2, 3, 3, 3, 3, 3, 3,
        3, 3, 3, 3, 3, 4, 4, 4, 4, 4, 4, 4, 5, 5, 5, 5, 5, 6, 7, 7, 7, 7, 7,
        7, 7, 7, 8, 8, 8, 8, 8, 8, 9, 9, 9, 9, 10, 10, 10, 11, 11, 11, 11,
        11, 11, 11, 11, 11, 12, 12, 12, 12, 12, 12, 12, 12, 12, 12, 12, 13,
        13, 13, 13, 13, 13, 13, 13, 13, 13, 13, 14, 14, 14, 14, 15, 15, 15),
    1: (0, 0, 0, 0, 1, 1, 1, 1, 1, 1, 1, 1, 1, 1, 2, 2, 2, 2, 2, 2, 3, 3, 3,
        3, 3, 4, 4, 4, 4, 4, 4, 4, 4, 4, 4, 4, 5, 5, 6, 6, 6, 6, 6, 6, 6, 6,
        7, 7, 7, 7, 7, 7, 7, 8, 8, 8, 8, 8, 8, 8, 9, 9, 9, 9, 9, 9, 9, 10,
        10, 10, 10, 10, 10, 10, 10, 10, 11, 11, 12, 12, 12, 12, 12, 12, 12,
        13, 13, 13, 13, 13, 14, 14, 14, 14, 14, 14, 15, 15, 15, 15),
    2: (0, 0, 0, 0, 1, 1, 1, 1, 1, 2, 2, 2, 2, 2, 2, 2, 2, 3, 3, 3, 3, 3, 3,
        3, 4, 4, 4, 4, 5, 5, 5, 5, 5, 5, 5, 5, 5, 5, 6, 6, 6, 6, 6, 7, 7, 7,
        7, 7, 8, 8, 8, 8, 8, 9, 9, 9, 9, 9, 9, 10, 10, 10, 10, 10, 10, 10,
        10, 11, 11, 11, 11, 11, 12, 12, 12, 12, 12, 13, 13, 13, 13, 13, 13,
        13, 13, 13, 13, 14, 14, 14, 14, 14, 14, 15, 15, 15, 15, 15, 15, 15),
}


def _sel_batches(call_id):
    return jnp.array(_SEL[call_id], dtype=jnp.int32)


def _sc_body(x_head, x_face, x_body, idx, o_head, o_face, o_body,
             slab6, slab21, out7, out22, idxv):
    wid = lax.axis_index("c") * 16 + lax.axis_index("s")

    def one_head(k, row, x_hbm, out_hbm, slab, outv, n_ch, lmks):
        @pl.when(wid == k)
        def _():
            pltpu.sync_copy(x_hbm, slab)
            pltpu.sync_copy(idx.at[pl.ds(row * PAD, PAD)], idxv)
            for g in range(PAD // LANES):
                bvec = idxv[pl.ds(g * LANES, LANES)]
                ivec = lax.iota(jnp.int32, LANES) + g * LANES
                ch = [plsc.load_gather(
                          slab, [bvec, ivec + SHIFT,
                                 jnp.full((LANES,), c, jnp.int32)])
                      for c in range(n_ch)]
                cols = [bvec.astype(jnp.float32),
                        ch[0] - 0.5 * ch[2], ch[1] - 0.5 * ch[3],
                        ch[0] + 0.5 * ch[2], ch[1] + 0.5 * ch[3],
                        jnp.zeros((LANES,), jnp.float32),
                        ch[4] * ch[5]]
                cols += [ch[c] for c in lmks]
                for j, col in enumerate(cols):
                    plsc.store_scatter(
                        outv, [ivec, jnp.full((LANES,), j, jnp.int32)], col)
            pltpu.sync_copy(outv.at[pl.ds(0, MAX_OBJ), :], out_hbm)

    one_head(0, 0, x_head, o_head, slab6, out7, 6, ())
    one_head(16, 1, x_face, o_face, slab21, out22, 21, LMK_CH)
    one_head(1, 2, x_body, o_body, slab6, out7, 6, ())


@jax.jit
def kernel(IDetectBody, IDetectHead, IKeypoint):
    idx = jnp.stack([_sel_batches(1), _sel_batches(2), _sel_batches(0)])
    idx = jnp.pad(idx, ((0, 0), (0, PAD - MAX_OBJ))).reshape(-1)

    sb = lax.slice_in_dim(IDetectBody, BASE, BASE + SLAB, axis=1)
    sh = lax.slice_in_dim(IDetectHead, BASE, BASE + SLAB, axis=1)
    sf = lax.slice_in_dim(IKeypoint, BASE, BASE + SLAB, axis=1)

    f32 = jnp.float32
    call = pl.kernel(
        _sc_body,
        out_type=(jax.ShapeDtypeStruct((MAX_OBJ, 7), f32),
                  jax.ShapeDtypeStruct((MAX_OBJ, 22), f32),
                  jax.ShapeDtypeStruct((MAX_OBJ, 7), f32)),
        mesh=plsc.VectorSubcoreMesh(core_axis_name="c", subcore_axis_name="s"),
        compiler_params=pltpu.CompilerParams(needs_layout_passes=False,
                                             use_tc_tiling_on_sc=False),
        scratch_types=[
            pltpu.VMEM((B, SLAB, 6), f32),
            pltpu.VMEM((B, SLAB, 21), f32),
            pltpu.VMEM((PAD, 7), f32),
            pltpu.VMEM((PAD, 22), f32),
            pltpu.VMEM((PAD,), jnp.int32),
        ],
    )
    return call(sh, sf, sb, idx)
